# baseline (device time: 55018 ns/iter reference)
import jax
import jax.numpy as jnp
from jax import lax
from jax.experimental import pallas as pl
from jax.experimental.pallas import tpu as pltpu


def kernel(O, Wo):
    B, S_full, H_loc, D = O.shape
    K = H_loc * D
    N = Wo.shape[1]
    S_half = S_full // 2

    O2 = O.reshape(B, S_full, K)

    def body(o_ref, w_ref, out_ref, send_ref, recv_ref, send_sems, recv_sems):
        my_x = lax.axis_index("x")
        my_y = lax.axis_index("y")
        peer = (my_x, 1 - my_y)

        barrier = pltpu.get_barrier_semaphore()
        pl.semaphore_signal(barrier, inc=1, device_id=peer,
                            device_id_type=pl.DeviceIdType.MESH)
        pl.semaphore_wait(barrier, 1)

        peer_s0 = (1 - my_y) * S_half
        my_s0 = my_y * S_half

        rdmas = []
        for b in range(B):
            send_ref[b] = jnp.dot(
                o_ref[b, pl.ds(peer_s0, S_half), :], w_ref[...],
                preferred_element_type=jnp.float32)
            rdma = pltpu.make_async_remote_copy(
                src_ref=send_ref.at[b],
                dst_ref=recv_ref.at[b],
                send_sem=send_sems.at[b],
                recv_sem=recv_sems.at[b],
                device_id=peer,
                device_id_type=pl.DeviceIdType.MESH,
            )
            rdma.start()
            rdmas.append(rdma)

        for b in range(B):
            out_ref[b] = jnp.dot(
                o_ref[b, pl.ds(my_s0, S_half), :], w_ref[...],
                preferred_element_type=jnp.float32)

        for b in range(B):
            rdmas[b].wait()
            out_ref[b] = out_ref[b] + recv_ref[b]

    return pl.pallas_call(
        body,
        out_shape=jax.ShapeDtypeStruct((B, S_half, N), jnp.float32),
        in_specs=[
            pl.BlockSpec(memory_space=pltpu.VMEM),
            pl.BlockSpec(memory_space=pltpu.VMEM),
        ],
        out_specs=pl.BlockSpec(memory_space=pltpu.VMEM),
        scratch_shapes=[
            pltpu.VMEM((B, S_half, N), jnp.float32),
            pltpu.VMEM((B, S_half, N), jnp.float32),
            pltpu.SemaphoreType.DMA((B,)),
            pltpu.SemaphoreType.DMA((B,)),
        ],
        compiler_params=pltpu.CompilerParams(collective_id=0),
    )(O2, Wo)


# device time: 37774 ns/iter; 1.4565x vs baseline; 1.4565x over previous
import jax
import jax.numpy as jnp
from jax import lax
from jax.experimental import pallas as pl
from jax.experimental.pallas import tpu as pltpu


def kernel(O, Wo):
    B, S_full, H_loc, D = O.shape
    K = H_loc * D
    N = Wo.shape[1]
    S_half = S_full // 2
    NH = N // 2
    NC = 256
    CPB = NH // NC
    C = B * CPB

    O2 = O.reshape(B, S_full, K)

    def body(o_ref, w_ref, out_ref, ysend, yrecv, xrecv,
             ysend_sems, yrecv_sems, fsend_sems, xrecv_sems):
        my_x = lax.axis_index("x")
        my_y = lax.axis_index("y")
        ypeer = (my_x, 1 - my_y)
        xpeer = (1 - my_x, my_y)

        barrier = pltpu.get_barrier_semaphore()
        for p in (ypeer, xpeer):
            pl.semaphore_signal(barrier, inc=1, device_id=p,
                                device_id_type=pl.DeviceIdType.MESH)
        pl.semaphore_wait(barrier, 2)

        peer_s0 = (1 - my_y) * S_half
        my_s0 = my_y * S_half
        nbase = my_x * NH
        obase = (1 - my_x) * NH

        y_rdmas = []
        for b in range(B):
            for q in range(CPB):
                c = b * CPB + q
                ysend[c] = jnp.dot(
                    o_ref[b, pl.ds(peer_s0, S_half), :],
                    w_ref[:, pl.ds(nbase + q * NC, NC)],
                    preferred_element_type=jnp.float32)
                r = pltpu.make_async_remote_copy(
                    src_ref=ysend.at[c], dst_ref=yrecv.at[c],
                    send_sem=ysend_sems.at[c], recv_sem=yrecv_sems.at[c],
                    device_id=ypeer, device_id_type=pl.DeviceIdType.MESH)
                r.start()
                y_rdmas.append(r)

        f_rdmas = []
        for b in range(B):
            for q in range(CPB):
                c = b * CPB + q
                out_ref[b, :, pl.ds(nbase + q * NC, NC)] = jnp.dot(
                    o_ref[b, pl.ds(my_s0, S_half), :],
                    w_ref[:, pl.ds(nbase + q * NC, NC)],
                    preferred_element_type=jnp.float32)
                out_ref[b, :, pl.ds(obase + q * NC, NC)] = jnp.dot(
                    o_ref[b, pl.ds(my_s0, S_half), :],
                    w_ref[:, pl.ds(obase + q * NC, NC)],
                    preferred_element_type=jnp.float32)
                y_rdmas[c].wait_recv()
                f = pltpu.make_async_remote_copy(
                    src_ref=yrecv.at[c], dst_ref=xrecv.at[c],
                    send_sem=fsend_sems.at[c], recv_sem=xrecv_sems.at[c],
                    device_id=xpeer, device_id_type=pl.DeviceIdType.MESH)
                f.start()
                f_rdmas.append(f)
                out_ref[b, :, pl.ds(nbase + q * NC, NC)] = (
                    out_ref[b, :, pl.ds(nbase + q * NC, NC)] + yrecv[c])

        for b in range(B):
            for q in range(CPB):
                c = b * CPB + q
                f_rdmas[c].wait_recv()
                out_ref[b, :, pl.ds(obase + q * NC, NC)] = (
                    out_ref[b, :, pl.ds(obase + q * NC, NC)] + xrecv[c])

        for r in y_rdmas:
            r.wait_send()
        for r in f_rdmas:
            r.wait_send()

    return pl.pallas_call(
        body,
        out_shape=jax.ShapeDtypeStruct((B, S_half, N), jnp.float32),
        in_specs=[
            pl.BlockSpec(memory_space=pltpu.VMEM),
            pl.BlockSpec(memory_space=pltpu.VMEM),
        ],
        out_specs=pl.BlockSpec(memory_space=pltpu.VMEM),
        scratch_shapes=[
            pltpu.VMEM((C, S_half, NC), jnp.float32),
            pltpu.VMEM((C, S_half, NC), jnp.float32),
            pltpu.VMEM((C, S_half, NC), jnp.float32),
            pltpu.SemaphoreType.DMA((C,)),
            pltpu.SemaphoreType.DMA((C,)),
            pltpu.SemaphoreType.DMA((C,)),
            pltpu.SemaphoreType.DMA((C,)),
        ],
        compiler_params=pltpu.CompilerParams(collective_id=0),
    )(O2, Wo)


# device time: 26660 ns/iter; 2.0637x vs baseline; 1.4169x over previous
import jax
import jax.numpy as jnp
from jax import lax
from jax.experimental import pallas as pl
from jax.experimental.pallas import tpu as pltpu


def kernel(O, Wo):
    B, S_full, H_loc, D = O.shape
    K = H_loc * D
    N = Wo.shape[1]
    S_half = S_full // 2
    NH = N // 2
    NC = 256
    CPB = NH // NC
    C = B * CPB

    O2 = O.reshape(B, S_full, K).astype(jnp.bfloat16)
    Wb = Wo.astype(jnp.bfloat16)

    def body(o_ref, w_ref, out_ref, ysend, yrecv, xrecv,
             ysend_sems, yrecv_sems, fsend_sems, xrecv_sems):
        my_x = lax.axis_index("x")
        my_y = lax.axis_index("y")
        ypeer = (my_x, 1 - my_y)
        xpeer = (1 - my_x, my_y)

        barrier = pltpu.get_barrier_semaphore()
        for p in (ypeer, xpeer):
            pl.semaphore_signal(barrier, inc=1, device_id=p,
                                device_id_type=pl.DeviceIdType.MESH)
        pl.semaphore_wait(barrier, 2)

        peer_s0 = (1 - my_y) * S_half
        my_s0 = my_y * S_half
        nbase = my_x * NH
        obase = (1 - my_x) * NH

        y_rdmas = []
        for b in range(B):
            for q in range(CPB):
                c = b * CPB + q
                ysend[c] = jnp.dot(
                    o_ref[b, pl.ds(peer_s0, S_half), :],
                    w_ref[:, pl.ds(nbase + q * NC, NC)],
                    preferred_element_type=jnp.float32).astype(jnp.bfloat16)
                r = pltpu.make_async_remote_copy(
                    src_ref=ysend.at[c], dst_ref=yrecv.at[c],
                    send_sem=ysend_sems.at[c], recv_sem=yrecv_sems.at[c],
                    device_id=ypeer, device_id_type=pl.DeviceIdType.MESH)
                r.start()
                y_rdmas.append(r)

        f_rdmas = []
        for b in range(B):
            for q in range(CPB):
                c = b * CPB + q
                out_ref[b, :, pl.ds(nbase + q * NC, NC)] = jnp.dot(
                    o_ref[b, pl.ds(my_s0, S_half), :],
                    w_ref[:, pl.ds(nbase + q * NC, NC)],
                    preferred_element_type=jnp.float32)
                out_ref[b, :, pl.ds(obase + q * NC, NC)] = jnp.dot(
                    o_ref[b, pl.ds(my_s0, S_half), :],
                    w_ref[:, pl.ds(obase + q * NC, NC)],
                    preferred_element_type=jnp.float32)
                y_rdmas[c].wait_recv()
                f = pltpu.make_async_remote_copy(
                    src_ref=yrecv.at[c], dst_ref=xrecv.at[c],
                    send_sem=fsend_sems.at[c], recv_sem=xrecv_sems.at[c],
                    device_id=xpeer, device_id_type=pl.DeviceIdType.MESH)
                f.start()
                f_rdmas.append(f)
                out_ref[b, :, pl.ds(nbase + q * NC, NC)] = (
                    out_ref[b, :, pl.ds(nbase + q * NC, NC)]
                    + yrecv[c].astype(jnp.float32))

        for b in range(B):
            for q in range(CPB):
                c = b * CPB + q
                f_rdmas[c].wait_recv()
                out_ref[b, :, pl.ds(obase + q * NC, NC)] = (
                    out_ref[b, :, pl.ds(obase + q * NC, NC)]
                    + xrecv[c].astype(jnp.float32))

        for r in y_rdmas:
            r.wait_send()
        for r in f_rdmas:
            r.wait_send()

    return pl.pallas_call(
        body,
        out_shape=jax.ShapeDtypeStruct((B, S_half, N), jnp.float32),
        in_specs=[
            pl.BlockSpec(memory_space=pltpu.VMEM),
            pl.BlockSpec(memory_space=pltpu.VMEM),
        ],
        out_specs=pl.BlockSpec(memory_space=pltpu.VMEM),
        scratch_shapes=[
            pltpu.VMEM((C, S_half, NC), jnp.bfloat16),
            pltpu.VMEM((C, S_half, NC), jnp.bfloat16),
            pltpu.VMEM((C, S_half, NC), jnp.bfloat16),
            pltpu.SemaphoreType.DMA((C,)),
            pltpu.SemaphoreType.DMA((C,)),
            pltpu.SemaphoreType.DMA((C,)),
            pltpu.SemaphoreType.DMA((C,)),
        ],
        compiler_params=pltpu.CompilerParams(collective_id=0),
    )(O2, Wb)
